# baseline (speedup 1.0000x reference)
import functools

import jax
import jax.numpy as jnp
from jax import lax
from jax.experimental import pallas as pl
from jax.experimental.pallas import tpu as pltpu
from jax.experimental.pallas import tpu_sc as plsc

RESO = 128
PADDING = 0.1
HID = 32
CDIM = 32
NBLOCKS = 5
NSEG = RESO * RESO
B, T = 4, 50000

ROWS = 2048
WIN = 2000
NWIN = T // WIN
NVREG = WIN // 16
NEG = -3.4e38

_SC_MESH = plsc.VectorSubcoreMesh(core_axis_name="c", subcore_axis_name="s")
_SC_PARAMS = pltpu.CompilerParams(needs_layout_passes=False)


def _normalize_coordinate(p, plane):
    if plane == 'xz':
        xy = jnp.stack((p[..., 0], p[..., 2]), axis=-1)
    elif plane == 'xy':
        xy = jnp.stack((p[..., 0], p[..., 1]), axis=-1)
    else:
        xy = jnp.stack((p[..., 1], p[..., 2]), axis=-1)
    xy_new = xy / (1 + PADDING + 10e-4) + 0.5
    xy_new = jnp.where(xy_new >= 1.0, 1.0 - 10e-6, xy_new)
    xy_new = jnp.where(xy_new < 0.0, 0.0, xy_new)
    return xy_new


def _coordinate2index(x):
    xi = (x * RESO).astype(jnp.int32)
    return xi[..., 0] + RESO * xi[..., 1]


TB = T


def _wspec(shape):
    return pl.BlockSpec(shape, lambda b: (0,) * len(shape))


def _colspec(nf):
    return pl.BlockSpec((1, nf, TB), lambda b: (b, 0, 0))


def _block0_body(p_ref, wpt, bp, w0t, b0, w1t, b1, wst, o_ref):
    x = wpt[...] @ p_ref[0] + bp[...]
    h = w0t[...] @ jnp.maximum(x, 0.0) + b0[...]
    dx = w1t[...] @ jnp.maximum(h, 0.0) + b1[...]
    o_ref[0] = wst[...] @ x + dx


def _pallas_block0(p_fm, wpt, bp, w0t, b0, w1t, b1, wst):
    return pl.pallas_call(
        _block0_body,
        grid=(B,),
        in_specs=[
            _colspec(3),
            _wspec((2 * HID, 3)), _wspec((2 * HID, 1)),
            _wspec((HID, 2 * HID)), _wspec((HID, 1)),
            _wspec((HID, HID)), _wspec((HID, 1)),
            _wspec((HID, 2 * HID)),
        ],
        out_specs=_colspec(HID),
        out_shape=jax.ShapeDtypeStruct((B, HID, T), jnp.float32),
    )(p_fm, wpt, bp, w0t, b0, w1t, b1, wst)


def _blockmid_body(x1_ref, x2_ref, w0ta, w0tb, b0, w1t, b1, wsta, wstb,
                   o_ref):
    x1 = x1_ref[0]
    x2 = x2_ref[0]
    h = (w0ta[...] @ jnp.maximum(x1, 0.0)
         + w0tb[...] @ jnp.maximum(x2, 0.0) + b0[...])
    dx = w1t[...] @ jnp.maximum(h, 0.0) + b1[...]
    o_ref[0] = wsta[...] @ x1 + wstb[...] @ x2 + dx


def _pallas_blockmid(net_fm, pooled_fm, w0ta, w0tb, b0, w1t, b1, wsta, wstb):
    return pl.pallas_call(
        _blockmid_body,
        grid=(B,),
        in_specs=[
            _colspec(HID), _colspec(HID),
            _wspec((HID, HID)), _wspec((HID, HID)), _wspec((HID, 1)),
            _wspec((HID, HID)), _wspec((HID, 1)),
            _wspec((HID, HID)), _wspec((HID, HID)),
        ],
        out_specs=_colspec(HID),
        out_shape=jax.ShapeDtypeStruct((B, HID, T), jnp.float32),
    )(net_fm, pooled_fm, w0ta, w0tb, b0, w1t, b1, wsta, wstb)


def _blockc_body(x1_ref, x2_ref, w0ta, w0tb, b0, w1t, b1, wsta, wstb,
                 wct, bc, o_ref):
    x1 = x1_ref[0]
    x2 = x2_ref[0]
    h = (w0ta[...] @ jnp.maximum(x1, 0.0)
         + w0tb[...] @ jnp.maximum(x2, 0.0) + b0[...])
    dx = w1t[...] @ jnp.maximum(h, 0.0) + b1[...]
    net = wsta[...] @ x1 + wstb[...] @ x2 + dx
    o_ref[0] = wct[...] @ net + bc[...]


def _pallas_blockc(net_fm, pooled_fm, w0ta, w0tb, b0, w1t, b1, wsta, wstb,
                   wct, bc):
    return pl.pallas_call(
        _blockc_body,
        grid=(B,),
        in_specs=[
            _colspec(HID), _colspec(HID),
            _wspec((HID, HID)), _wspec((HID, HID)), _wspec((HID, 1)),
            _wspec((HID, HID)), _wspec((HID, 1)),
            _wspec((HID, HID)), _wspec((HID, HID)),
            _wspec((CDIM, HID)), _wspec((CDIM, 1)),
        ],
        out_specs=_colspec(CDIM),
        out_shape=jax.ShapeDtypeStruct((B, CDIM, T), jnp.float32),
    )(net_fm, pooled_fm, w0ta, w0tb, b0, w1t, b1, wsta, wstb, wct, bc)


def _scatter_max(acc, ix, v):
    cur = plsc.load_gather(acc, [ix])
    plsc.store_scatter(acc, [ix], jnp.maximum(cur, v))
    cur2 = plsc.load_gather(acc, [ix])
    m = cur2 < v

    def body(mm):
        c = plsc.load_gather(acc, [ix])
        plsc.store_scatter(acc, [ix], jnp.maximum(c, v), mask=mm)
        c2 = plsc.load_gather(acc, [ix])
        return mm & (c2 < v)

    lax.while_loop(lambda mm: jnp.any(mm), body, m)


NPAIR = (NWIN + 1) // 2


@functools.partial(
    pl.kernel, mesh=_SC_MESH, compiler_params=_SC_PARAMS,
    out_type=jax.ShapeDtypeStruct((B * HID * T,), jnp.float32),
    scratch_types=[
        pltpu.VMEM((NSEG,), jnp.float32),
        pltpu.VMEM((NSEG,), jnp.float32),
        pltpu.VMEM((NSEG,), jnp.float32),
        pltpu.VMEM((NSEG,), jnp.float32),
        pltpu.VMEM((NSEG,), jnp.float32),
        pltpu.VMEM((NSEG,), jnp.float32),
        pltpu.VMEM((WIN,), jnp.float32),
        pltpu.VMEM((WIN,), jnp.float32),
        pltpu.VMEM((WIN,), jnp.int32),
        pltpu.VMEM((WIN,), jnp.int32),
        pltpu.VMEM((WIN,), jnp.int32),
        pltpu.VMEM((WIN,), jnp.int32),
        pltpu.VMEM((WIN,), jnp.int32),
        pltpu.VMEM((WIN,), jnp.int32),
        pltpu.VMEM((WIN,), jnp.float32),
        pltpu.VMEM((WIN,), jnp.float32),
        pltpu.SemaphoreType.DMA,
        pltpu.SemaphoreType.DMA,
        pltpu.SemaphoreType.DMA,
        pltpu.SemaphoreType.DMA,
    ],
)
def _sc_pool(net_hbm, idx0_hbm, idx1_hbm, idx2_hbm, out_hbm,
             a0, a1, a2, a0o, a1o, a2o, vb0, vb1, i00, i10, i20,
             i01, i11, i21, ob0, ob1, smA0, smA1, smO0, smO1):
    w = lax.axis_index("s") * 2 + lax.axis_index("c")
    accs_e = (a0, a1, a2)
    accs_o = (a0o, a1o, a2o)
    vbufs = (vb0, vb1)
    ibufs = ((i00, i10, i20), (i01, i11, i21))
    obufs = (ob0, ob1)
    smA = (smA0, smA1)
    smO = (smO0, smO1)

    def per_batch(b, _):
        @plsc.parallel_loop(0, NSEG // 16, unroll=4)
        def initbody(i):
            z = jnp.full((16,), NEG, jnp.float32)
            a0[pl.ds(i * 16, 16)] = z
            a1[pl.ds(i * 16, 16)] = z
            a2[pl.ds(i * 16, 16)] = z
            a0o[pl.ds(i * 16, 16)] = z
            a1o[pl.ds(i * 16, 16)] = z
            a2o[pl.ds(i * 16, 16)] = z

        vbase = pl.multiple_of((b * HID + w) * T, 8)
        ibase = pl.multiple_of(b * T, 8)

        def vsrc(wi):
            off = pl.multiple_of(wi * WIN, 8)
            return net_hbm.at[pl.ds(vbase + off, WIN)]

        def isrcs(wi):
            off = pl.multiple_of(wi * WIN, 8)
            return (idx0_hbm.at[pl.ds(ibase + off, WIN)],
                    idx1_hbm.at[pl.ds(ibase + off, WIN)],
                    idx2_hbm.at[pl.ds(ibase + off, WIN)])

        def issueA(wi, s):
            pltpu.async_copy(vsrc(wi), vbufs[s], smA[s])
            for src, dst in zip(isrcs(wi), ibufs[s]):
                pltpu.async_copy(src, dst, smA[s])

        def waitA(wi, s):
            pltpu.make_async_copy(vsrc(wi), vbufs[s], smA[s]).wait()
            for src, dst in zip(isrcs(wi), ibufs[s]):
                pltpu.make_async_copy(src, dst, smA[s]).wait()

        issueA(0, 0)

        def gloopA(g, _):
            for s in (0, 1):
                wi = 2 * g + s

                @pl.when(wi + 1 < NWIN)
                def _():
                    issueA(wi + 1, 1 - s)

                @pl.when(wi < NWIN)
                def _():
                    waitA(wi, s)
                    vb = vbufs[s]
                    ibs = ibufs[s]

                    def one_vreg(sl, accs):
                        v = vb[sl]
                        fail = None
                        for acc, ib in zip(accs, ibs):
                            ix = ib[sl]
                            cnt, _ = plsc.scan_count(ix)
                            cur = plsc.load_gather(acc, [ix])
                            plsc.store_scatter(acc, [ix], jnp.maximum(cur, v))
                            f = cnt > 1
                            fail = f if fail is None else (fail | f)

                        @pl.when(jnp.any(fail))
                        def _():
                            for acc, ib in zip(accs, ibs):
                                _scatter_max(acc, ib[sl], v)

                    def vA(j, _):
                        one_vreg(pl.ds(j * 32, 16), accs_e)
                        one_vreg(pl.ds(j * 32 + 16, 16), accs_o)
                        return 0

                    lax.fori_loop(0, NVREG // 2, vA, 0)
                    one_vreg(pl.ds((NVREG - 1) * 16, 16), accs_e)
            return 0

        lax.fori_loop(0, NPAIR, gloopA, 0)

        def issueB(wi, s):
            for src, dst in zip(isrcs(wi), ibufs[s]):
                pltpu.async_copy(src, dst, smA[s])

        def waitB(wi, s):
            for src, dst in zip(isrcs(wi), ibufs[s]):
                pltpu.make_async_copy(src, dst, smA[s]).wait()

        def wait_out(wi, s):
            pltpu.make_async_copy(obufs[s], vsrc(wi), smO[s]).wait()

        issueB(0, 0)

        def gloopB(g, _):
            for s in (0, 1):
                wi = 2 * g + s

                @pl.when(wi + 1 < NWIN)
                def _():
                    issueB(wi + 1, 1 - s)

                @pl.when(wi < NWIN)
                def _():
                    waitB(wi, s)

                    @pl.when(wi >= 2)
                    def _():
                        wait_out(wi - 2, s)

                    ob = obufs[s]
                    ibs = ibufs[s]

                    @plsc.parallel_loop(0, NVREG, unroll=5)
                    def vB(j):
                        sl = pl.ds(j * 16, 16)
                        g0 = jnp.maximum(plsc.load_gather(a0, [ibs[0][sl]]),
                                         plsc.load_gather(a0o, [ibs[0][sl]]))
                        g1 = jnp.maximum(plsc.load_gather(a1, [ibs[1][sl]]),
                                         plsc.load_gather(a1o, [ibs[1][sl]]))
                        g2 = jnp.maximum(plsc.load_gather(a2, [ibs[2][sl]]),
                                         plsc.load_gather(a2o, [ibs[2][sl]]))
                        ob[sl] = g0 + g1 + g2
                    off = pl.multiple_of(wi * WIN, 8)
                    pltpu.async_copy(ob, out_hbm.at[pl.ds(vbase + off, WIN)],
                                     smO[s])
            return 0

        lax.fori_loop(0, NPAIR, gloopB, 0)
        pltpu.make_async_copy(
            obufs[1], vsrc(NWIN - 2), smO[1]).wait()
        pltpu.make_async_copy(
            obufs[0], vsrc(NWIN - 1), smO[0]).wait()
        return 0

    lax.fori_loop(0, B, per_batch, 0)


@functools.partial(
    pl.kernel, mesh=_SC_MESH, compiler_params=_SC_PARAMS,
    out_type=[jax.ShapeDtypeStruct((B * CDIM * NSEG,), jnp.float32)] * 3,
    scratch_types=[
        pltpu.VMEM((NSEG,), jnp.float32),
        pltpu.VMEM((NSEG,), jnp.float32),
        pltpu.VMEM((NSEG,), jnp.float32),
        pltpu.VMEM((NSEG,), jnp.float32),
        pltpu.VMEM((NSEG,), jnp.float32),
        pltpu.VMEM((NSEG,), jnp.float32),
        pltpu.VMEM((WIN,), jnp.float32),
        pltpu.VMEM((WIN,), jnp.float32),
        pltpu.VMEM((WIN,), jnp.int32),
        pltpu.VMEM((WIN,), jnp.int32),
        pltpu.VMEM((WIN,), jnp.int32),
        pltpu.VMEM((WIN,), jnp.int32),
        pltpu.VMEM((WIN,), jnp.int32),
        pltpu.VMEM((WIN,), jnp.int32),
        pltpu.SemaphoreType.DMA,
        pltpu.SemaphoreType.DMA,
    ],
)
def _sc_mean(c_hbm, idx0_hbm, idx1_hbm, idx2_hbm,
             o0_hbm, o1_hbm, o2_hbm,
             s0, s1, s2, c0, c1, c2,
             vb0, vb1, i00, i10, i20, i01, i11, i21, smA0, smA1):
    w = lax.axis_index("s") * 2 + lax.axis_index("c")
    vbufs = (vb0, vb1)
    ibufs = ((i00, i10, i20), (i01, i11, i21))
    smA = (smA0, smA1)

    def per_batch(b, _):
        @plsc.parallel_loop(0, NSEG // 16, unroll=4)
        def initbody(i):
            z = jnp.zeros((16,), jnp.float32)
            s0[pl.ds(i * 16, 16)] = z
            s1[pl.ds(i * 16, 16)] = z
            s2[pl.ds(i * 16, 16)] = z
            c0[pl.ds(i * 16, 16)] = z
            c1[pl.ds(i * 16, 16)] = z
            c2[pl.ds(i * 16, 16)] = z

        vbase = pl.multiple_of((b * CDIM + w) * T, 8)
        ibase = pl.multiple_of(b * T, 8)

        def vsrc(wi):
            off = pl.multiple_of(wi * WIN, 8)
            return c_hbm.at[pl.ds(vbase + off, WIN)]

        def isrcs(wi):
            off = pl.multiple_of(wi * WIN, 8)
            return (idx0_hbm.at[pl.ds(ibase + off, WIN)],
                    idx1_hbm.at[pl.ds(ibase + off, WIN)],
                    idx2_hbm.at[pl.ds(ibase + off, WIN)])

        def issueA(wi, s):
            pltpu.async_copy(vsrc(wi), vbufs[s], smA[s])
            for src, dst in zip(isrcs(wi), ibufs[s]):
                pltpu.async_copy(src, dst, smA[s])

        def waitA(wi, s):
            pltpu.make_async_copy(vsrc(wi), vbufs[s], smA[s]).wait()
            for src, dst in zip(isrcs(wi), ibufs[s]):
                pltpu.make_async_copy(src, dst, smA[s]).wait()

        issueA(0, 0)

        def gloopA(g, _):
            for s in (0, 1):
                wi = 2 * g + s

                @pl.when(wi + 1 < NWIN)
                def _():
                    issueA(wi + 1, 1 - s)

                @pl.when(wi < NWIN)
                def _():
                    waitA(wi, s)
                    vb = vbufs[s]
                    ibs = ibufs[s]

                    @plsc.parallel_loop(0, NVREG, unroll=5)
                    def vA(j):
                        sl = pl.ds(j * 16, 16)
                        v = vb[sl]
                        ones = jnp.ones((16,), jnp.float32)
                        plsc.addupdate_scatter(s0, [ibs[0][sl]], v)
                        plsc.addupdate_scatter(c0, [ibs[0][sl]], ones)
                        plsc.addupdate_scatter(s1, [ibs[1][sl]], v)
                        plsc.addupdate_scatter(c1, [ibs[1][sl]], ones)
                        plsc.addupdate_scatter(s2, [ibs[2][sl]], v)
                        plsc.addupdate_scatter(c2, [ibs[2][sl]], ones)
            return 0

        lax.fori_loop(0, NPAIR, gloopA, 0)

        @plsc.parallel_loop(0, NSEG // 16, unroll=4)
        def divbody(i):
            s = pl.ds(i * 16, 16)
            one = jnp.float32(1.0)
            s0[s] = s0[s] / jnp.maximum(c0[s], one)
            s1[s] = s1[s] / jnp.maximum(c1[s], one)
            s2[s] = s2[s] / jnp.maximum(c2[s], one)

        obase = pl.multiple_of((b * CDIM + w) * NSEG, 8)
        pltpu.sync_copy(s0, o0_hbm.at[pl.ds(obase, NSEG)])
        pltpu.sync_copy(s1, o1_hbm.at[pl.ds(obase, NSEG)])
        pltpu.sync_copy(s2, o2_hbm.at[pl.ds(obase, NSEG)])
        return 0

    lax.fori_loop(0, B, per_batch, 0)


def kernel(p, W_pos, b_pos, W0, b0, W1, b1, Ws, W_c, b_c):
    idx0 = _coordinate2index(_normalize_coordinate(p, 'xz')).ravel()
    idx1 = _coordinate2index(_normalize_coordinate(p, 'xy')).ravel()
    idx2 = _coordinate2index(_normalize_coordinate(p, 'yz')).ravel()

    p_fm = p.transpose(0, 2, 1)
    wpt = W_pos.T
    bp = b_pos[:, None]

    def col(v):
        return v[:, None]

    net_fm = _pallas_block0(p_fm, wpt, bp,
                            W0[0].T, col(b0[0]), W1[0].T, col(b1[0]), Ws[0].T)

    for i in range(1, NBLOCKS):
        pooled_fm = _sc_pool(net_fm.ravel(), idx0, idx1, idx2)
        pooled_fm = pooled_fm.reshape(B, HID, T)
        w0t = W0[i].T
        wst = Ws[i].T
        args = (net_fm, pooled_fm,
                w0t[:, :HID], w0t[:, HID:], col(b0[i]),
                W1[i].T, col(b1[i]), wst[:, :HID], wst[:, HID:])
        if i < NBLOCKS - 1:
            net_fm = _pallas_blockmid(*args)
        else:
            c_fm = _pallas_blockc(*args, W_c.T, b_c[:, None])

    o0, o1, o2 = _sc_mean(c_fm.ravel(), idx0, idx1, idx2)
    return (o0.reshape(B, CDIM, RESO, RESO),
            o1.reshape(B, CDIM, RESO, RESO),
            o2.reshape(B, CDIM, RESO, RESO))


# speedup vs baseline: 1.0291x; 1.0291x over previous
import functools

import jax
import jax.numpy as jnp
from jax import lax
from jax.experimental import pallas as pl
from jax.experimental.pallas import tpu as pltpu
from jax.experimental.pallas import tpu_sc as plsc

RESO = 128
PADDING = 0.1
HID = 32
CDIM = 32
NBLOCKS = 5
NSEG = RESO * RESO
B, T = 4, 50000

ROWS = 2048
WIN = 2000
NWIN = T // WIN
NVREG = WIN // 16
NEG = -3.4e38

_SC_MESH = plsc.VectorSubcoreMesh(core_axis_name="c", subcore_axis_name="s")
_SC_PARAMS = pltpu.CompilerParams(needs_layout_passes=False)


def _normalize_coordinate(p, plane):
    if plane == 'xz':
        xy = jnp.stack((p[..., 0], p[..., 2]), axis=-1)
    elif plane == 'xy':
        xy = jnp.stack((p[..., 0], p[..., 1]), axis=-1)
    else:
        xy = jnp.stack((p[..., 1], p[..., 2]), axis=-1)
    xy_new = xy / (1 + PADDING + 10e-4) + 0.5
    xy_new = jnp.where(xy_new >= 1.0, 1.0 - 10e-6, xy_new)
    xy_new = jnp.where(xy_new < 0.0, 0.0, xy_new)
    return xy_new


def _coordinate2index(x):
    xi = (x * RESO).astype(jnp.int32)
    return xi[..., 0] + RESO * xi[..., 1]


TB = T


def _wspec(shape):
    return pl.BlockSpec(shape, lambda b: (0,) * len(shape))


def _colspec(nf):
    return pl.BlockSpec((1, nf, TB), lambda b: (b, 0, 0))


def _block0_body(p_ref, wpt, bp, w0t, b0, w1t, b1, wst, o_ref):
    x = wpt[...] @ p_ref[0] + bp[...]
    h = w0t[...] @ jnp.maximum(x, 0.0) + b0[...]
    dx = w1t[...] @ jnp.maximum(h, 0.0) + b1[...]
    o_ref[0] = wst[...] @ x + dx


def _pallas_block0(p_fm, wpt, bp, w0t, b0, w1t, b1, wst):
    return pl.pallas_call(
        _block0_body,
        grid=(B,),
        in_specs=[
            _colspec(3),
            _wspec((2 * HID, 3)), _wspec((2 * HID, 1)),
            _wspec((HID, 2 * HID)), _wspec((HID, 1)),
            _wspec((HID, HID)), _wspec((HID, 1)),
            _wspec((HID, 2 * HID)),
        ],
        out_specs=_colspec(HID),
        out_shape=jax.ShapeDtypeStruct((B, HID, T), jnp.float32),
    )(p_fm, wpt, bp, w0t, b0, w1t, b1, wst)


def _blockmid_body(x1_ref, x2_ref, w0ta, w0tb, b0, w1t, b1, wsta, wstb,
                   o_ref):
    x1 = x1_ref[0]
    x2 = x2_ref[0]
    h = (w0ta[...] @ jnp.maximum(x1, 0.0)
         + w0tb[...] @ jnp.maximum(x2, 0.0) + b0[...])
    dx = w1t[...] @ jnp.maximum(h, 0.0) + b1[...]
    o_ref[0] = wsta[...] @ x1 + wstb[...] @ x2 + dx


def _pallas_blockmid(net_fm, pooled_fm, w0ta, w0tb, b0, w1t, b1, wsta, wstb):
    return pl.pallas_call(
        _blockmid_body,
        grid=(B,),
        in_specs=[
            _colspec(HID), _colspec(HID),
            _wspec((HID, HID)), _wspec((HID, HID)), _wspec((HID, 1)),
            _wspec((HID, HID)), _wspec((HID, 1)),
            _wspec((HID, HID)), _wspec((HID, HID)),
        ],
        out_specs=_colspec(HID),
        out_shape=jax.ShapeDtypeStruct((B, HID, T), jnp.float32),
    )(net_fm, pooled_fm, w0ta, w0tb, b0, w1t, b1, wsta, wstb)


def _blockc_body(x1_ref, x2_ref, w0ta, w0tb, b0, w1t, b1, wsta, wstb,
                 wct, bc, o_ref):
    x1 = x1_ref[0]
    x2 = x2_ref[0]
    h = (w0ta[...] @ jnp.maximum(x1, 0.0)
         + w0tb[...] @ jnp.maximum(x2, 0.0) + b0[...])
    dx = w1t[...] @ jnp.maximum(h, 0.0) + b1[...]
    net = wsta[...] @ x1 + wstb[...] @ x2 + dx
    o_ref[0] = wct[...] @ net + bc[...]


def _pallas_blockc(net_fm, pooled_fm, w0ta, w0tb, b0, w1t, b1, wsta, wstb,
                   wct, bc):
    return pl.pallas_call(
        _blockc_body,
        grid=(B,),
        in_specs=[
            _colspec(HID), _colspec(HID),
            _wspec((HID, HID)), _wspec((HID, HID)), _wspec((HID, 1)),
            _wspec((HID, HID)), _wspec((HID, 1)),
            _wspec((HID, HID)), _wspec((HID, HID)),
            _wspec((CDIM, HID)), _wspec((CDIM, 1)),
        ],
        out_specs=_colspec(CDIM),
        out_shape=jax.ShapeDtypeStruct((B, CDIM, T), jnp.float32),
    )(net_fm, pooled_fm, w0ta, w0tb, b0, w1t, b1, wsta, wstb, wct, bc)


def _scatter_max(acc, ix, v):
    cur = plsc.load_gather(acc, [ix])
    plsc.store_scatter(acc, [ix], jnp.maximum(cur, v))
    cur2 = plsc.load_gather(acc, [ix])
    m = cur2 < v

    def body(mm):
        c = plsc.load_gather(acc, [ix])
        plsc.store_scatter(acc, [ix], jnp.maximum(c, v), mask=mm)
        c2 = plsc.load_gather(acc, [ix])
        return mm & (c2 < v)

    lax.while_loop(lambda mm: jnp.any(mm), body, m)


NPAIR = (NWIN + 1) // 2


@functools.partial(
    pl.kernel, mesh=_SC_MESH, compiler_params=_SC_PARAMS,
    out_type=jax.ShapeDtypeStruct((B * HID * T,), jnp.float32),
    scratch_types=[
        pltpu.VMEM((NSEG,), jnp.float32),
        pltpu.VMEM((NSEG,), jnp.float32),
        pltpu.VMEM((NSEG,), jnp.float32),
        pltpu.VMEM((WIN,), jnp.float32),
        pltpu.VMEM((WIN,), jnp.float32),
        pltpu.VMEM((WIN,), jnp.int32),
        pltpu.VMEM((WIN,), jnp.int32),
        pltpu.VMEM((WIN,), jnp.int32),
        pltpu.VMEM((WIN,), jnp.int32),
        pltpu.VMEM((WIN,), jnp.int32),
        pltpu.VMEM((WIN,), jnp.int32),
        pltpu.VMEM((WIN,), jnp.float32),
        pltpu.VMEM((WIN,), jnp.float32),
        pltpu.SemaphoreType.DMA,
        pltpu.SemaphoreType.DMA,
        pltpu.SemaphoreType.DMA,
        pltpu.SemaphoreType.DMA,
    ],
)
def _sc_pool(net_hbm, idx0_hbm, idx1_hbm, idx2_hbm, out_hbm,
             a0, a1, a2, vb0, vb1, i00, i10, i20,
             i01, i11, i21, ob0, ob1, smA0, smA1, smO0, smO1):
    w = lax.axis_index("s") * 2 + lax.axis_index("c")
    accs = (a0, a1, a2)
    vbufs = (vb0, vb1)
    ibufs = ((i00, i10, i20), (i01, i11, i21))
    obufs = (ob0, ob1)
    smA = (smA0, smA1)
    smO = (smO0, smO1)

    def per_batch(b, _):
        @plsc.parallel_loop(0, NSEG // 16, unroll=4)
        def initbody(i):
            z = jnp.full((16,), NEG, jnp.float32)
            a0[pl.ds(i * 16, 16)] = z
            a1[pl.ds(i * 16, 16)] = z
            a2[pl.ds(i * 16, 16)] = z

        vbase = pl.multiple_of((b * HID + w) * T, 8)
        ibase = pl.multiple_of(b * T, 8)

        def vsrc(wi):
            off = pl.multiple_of(wi * WIN, 8)
            return net_hbm.at[pl.ds(vbase + off, WIN)]

        def isrcs(wi):
            off = pl.multiple_of(wi * WIN, 8)
            return (idx0_hbm.at[pl.ds(ibase + off, WIN)],
                    idx1_hbm.at[pl.ds(ibase + off, WIN)],
                    idx2_hbm.at[pl.ds(ibase + off, WIN)])

        def issueA(wi, s):
            pltpu.async_copy(vsrc(wi), vbufs[s], smA[s])
            for src, dst in zip(isrcs(wi), ibufs[s]):
                pltpu.async_copy(src, dst, smA[s])

        def waitA(wi, s):
            pltpu.make_async_copy(vsrc(wi), vbufs[s], smA[s]).wait()
            for src, dst in zip(isrcs(wi), ibufs[s]):
                pltpu.make_async_copy(src, dst, smA[s]).wait()

        issueA(0, 0)

        def gloopA(g, _):
            for s in (0, 1):
                wi = 2 * g + s

                @pl.when(wi + 1 < NWIN)
                def _():
                    issueA(wi + 1, 1 - s)

                @pl.when(wi < NWIN)
                def _():
                    waitA(wi, s)
                    vb = vbufs[s]
                    ibs = ibufs[s]

                    def one_vreg(sl):
                        v = vb[sl]
                        fail = None
                        for acc, ib in zip(accs, ibs):
                            ix = ib[sl]
                            cnt, _ = plsc.scan_count(ix)
                            cur = plsc.load_gather(acc, [ix])
                            plsc.store_scatter(acc, [ix], jnp.maximum(cur, v))
                            f = cnt > 1
                            fail = f if fail is None else (fail | f)

                        @pl.when(jnp.any(fail))
                        def _():
                            for acc, ib in zip(accs, ibs):
                                _scatter_max(acc, ib[sl], v)

                    def vA(j, _):
                        one_vreg(pl.ds(j * 32, 16))
                        one_vreg(pl.ds(j * 32 + 16, 16))
                        return 0

                    lax.fori_loop(0, NVREG // 2, vA, 0)
                    one_vreg(pl.ds((NVREG - 1) * 16, 16))
            return 0

        lax.fori_loop(0, NPAIR, gloopA, 0)

        def issueB(wi, s):
            for src, dst in zip(isrcs(wi), ibufs[s]):
                pltpu.async_copy(src, dst, smA[s])

        def waitB(wi, s):
            for src, dst in zip(isrcs(wi), ibufs[s]):
                pltpu.make_async_copy(src, dst, smA[s]).wait()

        def wait_out(wi, s):
            pltpu.make_async_copy(obufs[s], vsrc(wi), smO[s]).wait()

        issueB(0, 0)

        def gloopB(g, _):
            for s in (0, 1):
                wi = 2 * g + s

                @pl.when(wi + 1 < NWIN)
                def _():
                    issueB(wi + 1, 1 - s)

                @pl.when(wi < NWIN)
                def _():
                    waitB(wi, s)

                    @pl.when(wi >= 2)
                    def _():
                        wait_out(wi - 2, s)

                    ob = obufs[s]
                    ibs = ibufs[s]

                    @plsc.parallel_loop(0, NVREG, unroll=5)
                    def vB(j):
                        sl = pl.ds(j * 16, 16)
                        g0 = plsc.load_gather(a0, [ibs[0][sl]])
                        g1 = plsc.load_gather(a1, [ibs[1][sl]])
                        g2 = plsc.load_gather(a2, [ibs[2][sl]])
                        ob[sl] = g0 + g1 + g2
                    off = pl.multiple_of(wi * WIN, 8)
                    pltpu.async_copy(ob, out_hbm.at[pl.ds(vbase + off, WIN)],
                                     smO[s])
            return 0

        lax.fori_loop(0, NPAIR, gloopB, 0)
        pltpu.make_async_copy(
            obufs[1], vsrc(NWIN - 2), smO[1]).wait()
        pltpu.make_async_copy(
            obufs[0], vsrc(NWIN - 1), smO[0]).wait()
        return 0

    lax.fori_loop(0, B, per_batch, 0)


@functools.partial(
    pl.kernel, mesh=_SC_MESH, compiler_params=_SC_PARAMS,
    out_type=[jax.ShapeDtypeStruct((B * CDIM * NSEG,), jnp.float32)] * 3,
    scratch_types=[
        pltpu.VMEM((NSEG,), jnp.float32),
        pltpu.VMEM((NSEG,), jnp.float32),
        pltpu.VMEM((NSEG,), jnp.float32),
        pltpu.VMEM((NSEG,), jnp.float32),
        pltpu.VMEM((NSEG,), jnp.float32),
        pltpu.VMEM((NSEG,), jnp.float32),
        pltpu.VMEM((WIN,), jnp.float32),
        pltpu.VMEM((WIN,), jnp.float32),
        pltpu.VMEM((WIN,), jnp.int32),
        pltpu.VMEM((WIN,), jnp.int32),
        pltpu.VMEM((WIN,), jnp.int32),
        pltpu.VMEM((WIN,), jnp.int32),
        pltpu.VMEM((WIN,), jnp.int32),
        pltpu.VMEM((WIN,), jnp.int32),
        pltpu.SemaphoreType.DMA,
        pltpu.SemaphoreType.DMA,
    ],
)
def _sc_mean(c_hbm, idx0_hbm, idx1_hbm, idx2_hbm,
             o0_hbm, o1_hbm, o2_hbm,
             s0, s1, s2, c0, c1, c2,
             vb0, vb1, i00, i10, i20, i01, i11, i21, smA0, smA1):
    w = lax.axis_index("s") * 2 + lax.axis_index("c")
    vbufs = (vb0, vb1)
    ibufs = ((i00, i10, i20), (i01, i11, i21))
    smA = (smA0, smA1)

    def per_batch(b, _):
        @plsc.parallel_loop(0, NSEG // 16, unroll=4)
        def initbody(i):
            z = jnp.zeros((16,), jnp.float32)
            s0[pl.ds(i * 16, 16)] = z
            s1[pl.ds(i * 16, 16)] = z
            s2[pl.ds(i * 16, 16)] = z
            c0[pl.ds(i * 16, 16)] = z
            c1[pl.ds(i * 16, 16)] = z
            c2[pl.ds(i * 16, 16)] = z

        vbase = pl.multiple_of((b * CDIM + w) * T, 8)
        ibase = pl.multiple_of(b * T, 8)

        def vsrc(wi):
            off = pl.multiple_of(wi * WIN, 8)
            return c_hbm.at[pl.ds(vbase + off, WIN)]

        def isrcs(wi):
            off = pl.multiple_of(wi * WIN, 8)
            return (idx0_hbm.at[pl.ds(ibase + off, WIN)],
                    idx1_hbm.at[pl.ds(ibase + off, WIN)],
                    idx2_hbm.at[pl.ds(ibase + off, WIN)])

        def issueA(wi, s):
            pltpu.async_copy(vsrc(wi), vbufs[s], smA[s])
            for src, dst in zip(isrcs(wi), ibufs[s]):
                pltpu.async_copy(src, dst, smA[s])

        def waitA(wi, s):
            pltpu.make_async_copy(vsrc(wi), vbufs[s], smA[s]).wait()
            for src, dst in zip(isrcs(wi), ibufs[s]):
                pltpu.make_async_copy(src, dst, smA[s]).wait()

        issueA(0, 0)

        def gloopA(g, _):
            for s in (0, 1):
                wi = 2 * g + s

                @pl.when(wi + 1 < NWIN)
                def _():
                    issueA(wi + 1, 1 - s)

                @pl.when(wi < NWIN)
                def _():
                    waitA(wi, s)
                    vb = vbufs[s]
                    ibs = ibufs[s]

                    @plsc.parallel_loop(0, NVREG, unroll=5)
                    def vA(j):
                        sl = pl.ds(j * 16, 16)
                        v = vb[sl]
                        ones = jnp.ones((16,), jnp.float32)
                        plsc.addupdate_scatter(s0, [ibs[0][sl]], v)
                        plsc.addupdate_scatter(c0, [ibs[0][sl]], ones)
                        plsc.addupdate_scatter(s1, [ibs[1][sl]], v)
                        plsc.addupdate_scatter(c1, [ibs[1][sl]], ones)
                        plsc.addupdate_scatter(s2, [ibs[2][sl]], v)
                        plsc.addupdate_scatter(c2, [ibs[2][sl]], ones)
            return 0

        lax.fori_loop(0, NPAIR, gloopA, 0)

        @plsc.parallel_loop(0, NSEG // 16, unroll=4)
        def divbody(i):
            s = pl.ds(i * 16, 16)
            one = jnp.float32(1.0)
            s0[s] = s0[s] / jnp.maximum(c0[s], one)
            s1[s] = s1[s] / jnp.maximum(c1[s], one)
            s2[s] = s2[s] / jnp.maximum(c2[s], one)

        obase = pl.multiple_of((b * CDIM + w) * NSEG, 8)
        pltpu.sync_copy(s0, o0_hbm.at[pl.ds(obase, NSEG)])
        pltpu.sync_copy(s1, o1_hbm.at[pl.ds(obase, NSEG)])
        pltpu.sync_copy(s2, o2_hbm.at[pl.ds(obase, NSEG)])
        return 0

    lax.fori_loop(0, B, per_batch, 0)


def kernel(p, W_pos, b_pos, W0, b0, W1, b1, Ws, W_c, b_c):
    idx0 = _coordinate2index(_normalize_coordinate(p, 'xz')).ravel()
    idx1 = _coordinate2index(_normalize_coordinate(p, 'xy')).ravel()
    idx2 = _coordinate2index(_normalize_coordinate(p, 'yz')).ravel()

    p_fm = p.transpose(0, 2, 1)
    wpt = W_pos.T
    bp = b_pos[:, None]

    def col(v):
        return v[:, None]

    net_fm = _pallas_block0(p_fm, wpt, bp,
                            W0[0].T, col(b0[0]), W1[0].T, col(b1[0]), Ws[0].T)

    for i in range(1, NBLOCKS):
        pooled_fm = _sc_pool(net_fm.ravel(), idx0, idx1, idx2)
        pooled_fm = pooled_fm.reshape(B, HID, T)
        w0t = W0[i].T
        wst = Ws[i].T
        args = (net_fm, pooled_fm,
                w0t[:, :HID], w0t[:, HID:], col(b0[i]),
                W1[i].T, col(b1[i]), wst[:, :HID], wst[:, HID:])
        if i < NBLOCKS - 1:
            net_fm = _pallas_blockmid(*args)
        else:
            c_fm = _pallas_blockc(*args, W_c.T, b_c[:, None])

    o0, o1, o2 = _sc_mean(c_fm.ravel(), idx0, idx1, idx2)
    return (o0.reshape(B, CDIM, RESO, RESO),
            o1.reshape(B, CDIM, RESO, RESO),
            o2.reshape(B, CDIM, RESO, RESO))
